# Initial kernel scaffold; baseline (speedup 1.0000x reference)
import functools

import jax
import jax.numpy as jnp
from jax import lax
from jax.experimental import pallas as pl
from jax.experimental.pallas import tpu as pltpu
from jax.experimental.pallas import tpu_sc as plsc

N = 10000
E = 320000
D = 128
NP = 10240
NC = 2
NS = 16
NW = NC * NS
EPT = E // NW
CH = 80
NCHUNK = EPT // CH
RPT = NP // NS

_mesh = plsc.VectorSubcoreMesh(core_axis_name="c", subcore_axis_name="s")


@functools.partial(
    pl.kernel,
    out_type=jax.ShapeDtypeStruct((NC, NP), jnp.float32),
    mesh=_mesh,
    scratch_types=[
        pltpu.VMEM_SHARED((NP,), jnp.float32),
        pltpu.VMEM((CH,), jnp.int32),
        pltpu.VMEM((CH,), jnp.float32),
        pltpu.VMEM((RPT,), jnp.float32),
    ],
)
def _deg_kernel(src, dst, s, out, acc, idx_v, val_v, z_v):
    cid = lax.axis_index("c")
    sid = lax.axis_index("s")
    wid = sid * NC + cid

    def zb(i, _):
        z_v[pl.ds(i * 16, 16)] = jnp.zeros((16,), jnp.float32)
        return 0

    lax.fori_loop(0, RPT // 16, zb, 0)
    row0 = pl.multiple_of(sid * RPT, 8)
    pltpu.sync_copy(z_v, acc.at[pl.ds(row0, RPT)])
    plsc.subcore_barrier()

    base = wid * EPT

    def chunk(i, _):
        off = pl.multiple_of(base + i * CH, 8)
        pltpu.sync_copy(s.at[pl.ds(off, CH)], val_v)
        pltpu.sync_copy(dst.at[pl.ds(off, CH)], idx_v)
        pltpu.sync_copy(val_v, acc.at[idx_v], add=True)
        pltpu.sync_copy(src.at[pl.ds(off, CH)], idx_v)
        pltpu.sync_copy(val_v, acc.at[idx_v], add=True)
        return 0

    lax.fori_loop(0, NCHUNK, chunk, 0)
    plsc.subcore_barrier()
    pltpu.sync_copy(acc.at[pl.ds(row0, RPT)], out.at[cid, pl.ds(row0, RPT)])


@functools.partial(
    pl.kernel,
    out_type=jax.ShapeDtypeStruct((NC, NP, D), jnp.float32),
    mesh=_mesh,
    scratch_types=[
        pltpu.VMEM_SHARED((NP, D), jnp.float32),
        pltpu.VMEM((CH,), jnp.int32),
        pltpu.VMEM((CH,), jnp.int32),
        pltpu.VMEM((CH,), jnp.float32),
        pltpu.VMEM((CH, D), jnp.float32),
    ],
)
def _agg_kernel(h, src, dst, s, out, acc, si_v, di_v, sv_v, rows_v):
    cid = lax.axis_index("c")
    sid = lax.axis_index("s")
    wid = sid * NC + cid

    def zb(j, _):
        for c in range(D // 16):
            rows_v[j, pl.ds(c * 16, 16)] = jnp.zeros((16,), jnp.float32)
        return 0

    lax.fori_loop(0, CH, zb, 0)
    for p in range(RPT // CH):
        pltpu.sync_copy(rows_v, acc.at[pl.ds(sid * RPT + p * CH, CH)])
    plsc.subcore_barrier()

    base = wid * EPT

    def chunk(i, _):
        off = pl.multiple_of(base + i * CH, 8)
        pltpu.sync_copy(src.at[pl.ds(off, CH)], si_v)
        pltpu.sync_copy(dst.at[pl.ds(off, CH)], di_v)
        pltpu.sync_copy(s.at[pl.ds(off, CH)], sv_v)
        pltpu.sync_copy(h.at[si_v], rows_v)

        def scale(j, _):
            b = plsc.load_gather(sv_v, [jnp.full((16,), j, jnp.int32)])
            for c in range(D // 16):
                rows_v[j, pl.ds(c * 16, 16)] = rows_v[j, pl.ds(c * 16, 16)] * b
            return 0

        lax.fori_loop(0, CH, scale, 0)
        pltpu.sync_copy(rows_v, acc.at[di_v], add=True)
        return 0

    lax.fori_loop(0, NCHUNK, chunk, 0)
    plsc.subcore_barrier()
    row0 = pl.multiple_of(sid * RPT, 8)
    pltpu.sync_copy(acc.at[pl.ds(row0, RPT)], out.at[cid, pl.ds(row0, RPT)])


BM = 1024
GRID = NP // BM


def _tc1_body(x_ref, w_ref, degp_ref, h_ref, dinv_ref):
    deg = degp_ref[0, :] + degp_ref[1, :]
    dinv = lax.rsqrt(jnp.maximum(deg, 1e-12))
    h = jnp.dot(x_ref[...], w_ref[...], preferred_element_type=jnp.float32)
    h_ref[...] = h * dinv[:, None]
    dinv_ref[...] = dinv[:, None]


_tc1 = pl.pallas_call(
    _tc1_body,
    grid=(GRID,),
    in_specs=[
        pl.BlockSpec((BM, D), lambda i: (i, 0)),
        pl.BlockSpec((D, D), lambda i: (0, 0)),
        pl.BlockSpec((NC, BM), lambda i: (0, i)),
    ],
    out_specs=[
        pl.BlockSpec((BM, D), lambda i: (i, 0)),
        pl.BlockSpec((BM, 1), lambda i: (i, 0)),
    ],
    out_shape=[
        jax.ShapeDtypeStruct((NP, D), jnp.float32),
        jax.ShapeDtypeStruct((NP, 1), jnp.float32),
    ],
)


def _tc2_body(p_ref, dinv_ref, w_ref, x1_ref, h2_ref):
    x1 = (p_ref[0] + p_ref[1]) * dinv_ref[...]
    x1_ref[...] = x1
    h2_ref[...] = (
        jnp.dot(x1, w_ref[...], preferred_element_type=jnp.float32) * dinv_ref[...]
    )


_tc2 = pl.pallas_call(
    _tc2_body,
    grid=(GRID,),
    in_specs=[
        pl.BlockSpec((NC, BM, D), lambda i: (0, i, 0)),
        pl.BlockSpec((BM, 1), lambda i: (i, 0)),
        pl.BlockSpec((D, D), lambda i: (0, 0)),
    ],
    out_specs=[
        pl.BlockSpec((BM, D), lambda i: (i, 0)),
        pl.BlockSpec((BM, D), lambda i: (i, 0)),
    ],
    out_shape=[
        jax.ShapeDtypeStruct((NP, D), jnp.float32),
        jax.ShapeDtypeStruct((NP, D), jnp.float32),
    ],
)


def _tc3_body(x1_ref, q_ref, dinv_ref, o_ref):
    o_ref[...] = x1_ref[...] + (q_ref[0] + q_ref[1]) * dinv_ref[...]


_tc3 = pl.pallas_call(
    _tc3_body,
    grid=(GRID,),
    in_specs=[
        pl.BlockSpec((BM, D), lambda i: (i, 0)),
        pl.BlockSpec((NC, BM, D), lambda i: (0, i, 0)),
        pl.BlockSpec((BM, 1), lambda i: (i, 0)),
    ],
    out_specs=pl.BlockSpec((BM, D), lambda i: (i, 0)),
    out_shape=jax.ShapeDtypeStruct((NP, D), jnp.float32),
)


@jax.jit
def kernel(id_embedding, edge_index, s, W1, W2):
    src = edge_index[0]
    dst = edge_index[1]
    xp = jnp.zeros((NP, D), jnp.float32).at[:N].set(id_embedding)
    degp = _deg_kernel(src, dst, s)
    h1p, dinv = _tc1(xp, W1, degp)
    p = _agg_kernel(h1p, src, dst, s)
    x1, h2p = _tc2(p, dinv, W2)
    q = _agg_kernel(h2p, src, dst, s)
    out = _tc3(x1, q, dinv)
    return out[:N]


# speedup vs baseline: 7.9748x; 7.9748x over previous
import functools

import jax
import jax.numpy as jnp
from jax import lax
from jax.experimental import pallas as pl
from jax.experimental.pallas import tpu as pltpu
from jax.experimental.pallas import tpu_sc as plsc

N = 10000
E = 320000
D = 128
NP = 10240
NC = 2
NS = 16
NW = NC * NS
EPT = E // NW
CH = 80
NCHUNK = EPT // CH
RPT = NP // NS

_mesh = plsc.VectorSubcoreMesh(core_axis_name="c", subcore_axis_name="s")


@functools.partial(
    pl.kernel,
    out_type=jax.ShapeDtypeStruct((NC, NP), jnp.float32),
    mesh=_mesh,
    scratch_types=[
        pltpu.VMEM_SHARED((NP,), jnp.float32),
        pltpu.VMEM((CH,), jnp.int32),
        pltpu.VMEM((CH,), jnp.float32),
        pltpu.VMEM((RPT,), jnp.float32),
    ],
)
def _deg_kernel(src, dst, s, out, acc, idx_v, val_v, z_v):
    cid = lax.axis_index("c")
    sid = lax.axis_index("s")
    wid = sid * NC + cid

    def zb(i, _):
        z_v[pl.ds(i * 16, 16)] = jnp.zeros((16,), jnp.float32)
        return 0

    lax.fori_loop(0, RPT // 16, zb, 0)
    row0 = pl.multiple_of(sid * RPT, 8)
    pltpu.sync_copy(z_v, acc.at[pl.ds(row0, RPT)])
    plsc.subcore_barrier()

    base = wid * EPT

    def chunk(i, _):
        off = pl.multiple_of(base + i * CH, 8)
        pltpu.sync_copy(s.at[pl.ds(off, CH)], val_v)
        pltpu.sync_copy(dst.at[pl.ds(off, CH)], idx_v)
        pltpu.sync_copy(val_v, acc.at[idx_v], add=True)
        pltpu.sync_copy(src.at[pl.ds(off, CH)], idx_v)
        pltpu.sync_copy(val_v, acc.at[idx_v], add=True)
        return 0

    lax.fori_loop(0, NCHUNK, chunk, 0)
    plsc.subcore_barrier()
    pltpu.sync_copy(acc.at[pl.ds(row0, RPT)], out.at[cid, pl.ds(row0, RPT)])


@functools.partial(
    pl.kernel,
    out_type=jax.ShapeDtypeStruct((NC, NP, D), jnp.float32),
    mesh=_mesh,
    scratch_types=[
        pltpu.VMEM_SHARED((NP, D), jnp.float32),
        pltpu.VMEM((CH,), jnp.int32),
        pltpu.VMEM((CH,), jnp.int32),
        pltpu.VMEM((CH,), jnp.float32),
        pltpu.VMEM((CH, D), jnp.float32),
    ],
)
def _agg_kernel(h, src, dst, s, out, acc, si_v, di_v, sv_v, rows_v):
    cid = lax.axis_index("c")
    sid = lax.axis_index("s")
    wid = sid * NC + cid

    def zb(j, _):
        for c in range(D // 16):
            rows_v[j, pl.ds(c * 16, 16)] = jnp.zeros((16,), jnp.float32)
        return 0

    lax.fori_loop(0, CH, zb, 0)
    for p in range(RPT // CH):
        pltpu.sync_copy(rows_v, acc.at[pl.ds(sid * RPT + p * CH, CH)])
    plsc.subcore_barrier()

    base = wid * EPT

    def chunk(i, _):
        off = pl.multiple_of(base + i * CH, 8)
        pltpu.sync_copy(src.at[pl.ds(off, CH)], si_v)
        pltpu.sync_copy(dst.at[pl.ds(off, CH)], di_v)
        pltpu.sync_copy(s.at[pl.ds(off, CH)], sv_v)
        pltpu.sync_copy(h.at[si_v], rows_v)

        def scale(g, _):
            sv = sv_v[pl.ds(g * 16, 16)]
            for j2 in range(16):
                j = g * 16 + j2
                b = jnp.broadcast_to(sv[j2], (16,))
                for c in range(D // 16):
                    rows_v[j, pl.ds(c * 16, 16)] = rows_v[j, pl.ds(c * 16, 16)] * b
            return 0

        lax.fori_loop(0, CH // 16, scale, 0)
        pltpu.sync_copy(rows_v, acc.at[di_v], add=True)
        return 0

    lax.fori_loop(0, NCHUNK, chunk, 0)
    plsc.subcore_barrier()
    row0 = pl.multiple_of(sid * RPT, 8)
    pltpu.sync_copy(acc.at[pl.ds(row0, RPT)], out.at[cid, pl.ds(row0, RPT)])


BM = 1024
GRID = NP // BM


def _tc1_body(x_ref, w_ref, degp_ref, h_ref, dinv_ref):
    deg = degp_ref[0, :] + degp_ref[1, :]
    dinv = lax.rsqrt(jnp.maximum(deg, 1e-12))
    h = jnp.dot(x_ref[...], w_ref[...], preferred_element_type=jnp.float32)
    h_ref[...] = h * dinv[:, None]
    dinv_ref[...] = dinv[:, None]


_tc1 = pl.pallas_call(
    _tc1_body,
    grid=(GRID,),
    in_specs=[
        pl.BlockSpec((BM, D), lambda i: (i, 0)),
        pl.BlockSpec((D, D), lambda i: (0, 0)),
        pl.BlockSpec((NC, BM), lambda i: (0, i)),
    ],
    out_specs=[
        pl.BlockSpec((BM, D), lambda i: (i, 0)),
        pl.BlockSpec((BM, 1), lambda i: (i, 0)),
    ],
    out_shape=[
        jax.ShapeDtypeStruct((NP, D), jnp.float32),
        jax.ShapeDtypeStruct((NP, 1), jnp.float32),
    ],
)


def _tc2_body(p_ref, dinv_ref, w_ref, x1_ref, h2_ref):
    x1 = (p_ref[0] + p_ref[1]) * dinv_ref[...]
    x1_ref[...] = x1
    h2_ref[...] = (
        jnp.dot(x1, w_ref[...], preferred_element_type=jnp.float32) * dinv_ref[...]
    )


_tc2 = pl.pallas_call(
    _tc2_body,
    grid=(GRID,),
    in_specs=[
        pl.BlockSpec((NC, BM, D), lambda i: (0, i, 0)),
        pl.BlockSpec((BM, 1), lambda i: (i, 0)),
        pl.BlockSpec((D, D), lambda i: (0, 0)),
    ],
    out_specs=[
        pl.BlockSpec((BM, D), lambda i: (i, 0)),
        pl.BlockSpec((BM, D), lambda i: (i, 0)),
    ],
    out_shape=[
        jax.ShapeDtypeStruct((NP, D), jnp.float32),
        jax.ShapeDtypeStruct((NP, D), jnp.float32),
    ],
)


def _tc3_body(x1_ref, q_ref, dinv_ref, o_ref):
    o_ref[...] = x1_ref[...] + (q_ref[0] + q_ref[1]) * dinv_ref[...]


_tc3 = pl.pallas_call(
    _tc3_body,
    grid=(GRID,),
    in_specs=[
        pl.BlockSpec((BM, D), lambda i: (i, 0)),
        pl.BlockSpec((NC, BM, D), lambda i: (0, i, 0)),
        pl.BlockSpec((BM, 1), lambda i: (i, 0)),
    ],
    out_specs=pl.BlockSpec((BM, D), lambda i: (i, 0)),
    out_shape=jax.ShapeDtypeStruct((NP, D), jnp.float32),
)


@jax.jit
def kernel(id_embedding, edge_index, s, W1, W2):
    src = edge_index[0]
    dst = edge_index[1]
    xp = jnp.zeros((NP, D), jnp.float32).at[:N].set(id_embedding)
    degp = _deg_kernel(src, dst, s)
    h1p, dinv = _tc1(xp, W1, degp)
    p = _agg_kernel(h1p, src, dst, s)
    x1, h2p = _tc2(p, dinv, W2)
    q = _agg_kernel(h2p, src, dst, s)
    out = _tc3(x1, q, dinv)
    return out[:N]


# speedup vs baseline: 15.5288x; 1.9472x over previous
import functools

import jax
import jax.numpy as jnp
from jax import lax
from jax.experimental import pallas as pl
from jax.experimental.pallas import tpu as pltpu
from jax.experimental.pallas import tpu_sc as plsc

N = 10000
E = 320000
D = 128
NP = 10240
NC = 2
NS = 16
NW = NC * NS
EPT = E // NW
RPT = NP // NS

CH = 16
NCHUNK = EPT // CH
NBUF = 5
AHEAD = 3

DCH = 80
DNCHUNK = EPT // DCH

_mesh = plsc.VectorSubcoreMesh(core_axis_name="c", subcore_axis_name="s")


@functools.partial(
    pl.kernel,
    out_type=jax.ShapeDtypeStruct((NC, NP), jnp.float32),
    mesh=_mesh,
    scratch_types=[
        pltpu.VMEM_SHARED((NP,), jnp.float32),
        pltpu.VMEM((DCH,), jnp.int32),
        pltpu.VMEM((DCH,), jnp.float32),
        pltpu.VMEM((RPT,), jnp.float32),
    ],
)
def _deg_kernel(src, dst, s, out, acc, idx_v, val_v, z_v):
    cid = lax.axis_index("c")
    sid = lax.axis_index("s")
    wid = sid * NC + cid

    def zb(i, _):
        z_v[pl.ds(i * 16, 16)] = jnp.zeros((16,), jnp.float32)
        return 0

    lax.fori_loop(0, RPT // 16, zb, 0)
    row0 = pl.multiple_of(sid * RPT, 8)
    pltpu.sync_copy(z_v, acc.at[pl.ds(row0, RPT)])
    plsc.subcore_barrier()

    base = wid * EPT

    def chunk(i, _):
        off = pl.multiple_of(base + i * DCH, 8)
        pltpu.sync_copy(s.at[pl.ds(off, DCH)], val_v)
        pltpu.sync_copy(dst.at[pl.ds(off, DCH)], idx_v)
        pltpu.sync_copy(val_v, acc.at[idx_v], add=True)
        pltpu.sync_copy(src.at[pl.ds(off, DCH)], idx_v)
        pltpu.sync_copy(val_v, acc.at[idx_v], add=True)
        return 0

    lax.fori_loop(0, DNCHUNK, chunk, 0)
    plsc.subcore_barrier()
    pltpu.sync_copy(acc.at[pl.ds(row0, RPT)], out.at[cid, pl.ds(row0, RPT)])


@functools.partial(
    pl.kernel,
    out_type=jax.ShapeDtypeStruct((NC, NP, D), jnp.float32),
    mesh=_mesh,
    scratch_types=[
        pltpu.VMEM_SHARED((NP, D), jnp.float32),
        pltpu.VMEM((EPT,), jnp.int32),
        pltpu.VMEM((EPT,), jnp.int32),
        pltpu.VMEM((EPT,), jnp.float32),
        [pltpu.VMEM((CH, D), jnp.float32) for _ in range(NBUF)],
        [pltpu.SemaphoreType.DMA for _ in range(NBUF)],
        [pltpu.SemaphoreType.DMA for _ in range(NBUF)],
    ],
)
def _agg_kernel(h, src3, dst3, s3, out, acc, si_v, di_v, sv_v, rows, gsem, ssem):
    cid = lax.axis_index("c")
    sid = lax.axis_index("s")
    wid = sid * NC + cid

    def zb(j, _):
        for c in range(D // 16):
            rows[0][j, pl.ds(c * 16, 16)] = jnp.zeros((16,), jnp.float32)
        return 0

    lax.fori_loop(0, CH, zb, 0)
    nz = RPT // CH
    for p in range(nz):
        pltpu.async_copy(
            rows[0], acc.at[pl.ds(sid * RPT + p * CH, CH)], gsem[p % NBUF]
        )
    for p in range(nz):
        pltpu.make_async_copy(
            rows[0], acc.at[pl.ds(sid * RPT + p * CH, CH)], gsem[p % NBUF]
        ).wait()

    pltpu.sync_copy(src3.at[wid], si_v)
    pltpu.sync_copy(dst3.at[wid], di_v)
    pltpu.sync_copy(s3.at[wid], sv_v)
    plsc.subcore_barrier()

    def start_gather(j, b):
        si = si_v[pl.ds(j * CH, CH)]
        pltpu.async_copy(h.at[si], rows[b], gsem[b])

    for j0 in range(AHEAD):
        start_gather(j0, j0)

    def outer(t, _):
        for p in range(NBUF):
            i = t * NBUF + p
            j = i + AHEAD
            jb = (p + AHEAD) % NBUF

            @pl.when(j < NCHUNK)
            def _():
                @pl.when(j >= NBUF)
                def _():
                    di = di_v[pl.ds(j * CH, CH)]
                    pltpu.make_async_copy(
                        rows[jb], acc.at[di], ssem[jb]
                    ).wait()

                start_gather(j, jb)

            si = si_v[pl.ds(i * CH, CH)]
            pltpu.make_async_copy(h.at[si], rows[p], gsem[p]).wait()

            sv = sv_v[pl.ds(i * CH, CH)]
            for j2 in range(CH):
                b = jnp.broadcast_to(sv[j2], (16,))
                for c in range(D // 16):
                    rows[p][j2, pl.ds(c * 16, 16)] = (
                        rows[p][j2, pl.ds(c * 16, 16)] * b
                    )

            di = di_v[pl.ds(i * CH, CH)]
            pltpu.async_copy(rows[p], acc.at[di], ssem[p], add=True)
        return 0

    lax.fori_loop(0, NCHUNK // NBUF, outer, 0)

    for p in range(NBUF):
        i_last = NCHUNK - NBUF + p
        di = di_v[pl.ds(i_last * CH, CH)]
        pltpu.make_async_copy(rows[p], acc.at[di], ssem[p]).wait()

    plsc.subcore_barrier()
    row0 = pl.multiple_of(sid * RPT, 8)
    pltpu.sync_copy(acc.at[pl.ds(row0, RPT)], out.at[cid, pl.ds(row0, RPT)])


BM = 1024
GRID = NP // BM


def _tc1_body(x_ref, w_ref, degp_ref, h_ref, dinv_ref):
    deg = degp_ref[0, :] + degp_ref[1, :]
    dinv = lax.rsqrt(jnp.maximum(deg, 1e-12))
    h = jnp.dot(x_ref[...], w_ref[...], preferred_element_type=jnp.float32)
    h_ref[...] = h * dinv[:, None]
    dinv_ref[...] = dinv[:, None]


_tc1 = pl.pallas_call(
    _tc1_body,
    grid=(GRID,),
    in_specs=[
        pl.BlockSpec((BM, D), lambda i: (i, 0)),
        pl.BlockSpec((D, D), lambda i: (0, 0)),
        pl.BlockSpec((NC, BM), lambda i: (0, i)),
    ],
    out_specs=[
        pl.BlockSpec((BM, D), lambda i: (i, 0)),
        pl.BlockSpec((BM, 1), lambda i: (i, 0)),
    ],
    out_shape=[
        jax.ShapeDtypeStruct((NP, D), jnp.float32),
        jax.ShapeDtypeStruct((NP, 1), jnp.float32),
    ],
)


def _tc2_body(p_ref, dinv_ref, w_ref, x1_ref, h2_ref):
    x1 = (p_ref[0] + p_ref[1]) * dinv_ref[...]
    x1_ref[...] = x1
    h2_ref[...] = (
        jnp.dot(x1, w_ref[...], preferred_element_type=jnp.float32) * dinv_ref[...]
    )


_tc2 = pl.pallas_call(
    _tc2_body,
    grid=(GRID,),
    in_specs=[
        pl.BlockSpec((NC, BM, D), lambda i: (0, i, 0)),
        pl.BlockSpec((BM, 1), lambda i: (i, 0)),
        pl.BlockSpec((D, D), lambda i: (0, 0)),
    ],
    out_specs=[
        pl.BlockSpec((BM, D), lambda i: (i, 0)),
        pl.BlockSpec((BM, D), lambda i: (i, 0)),
    ],
    out_shape=[
        jax.ShapeDtypeStruct((NP, D), jnp.float32),
        jax.ShapeDtypeStruct((NP, D), jnp.float32),
    ],
)


def _tc3_body(x1_ref, q_ref, dinv_ref, o_ref):
    o_ref[...] = x1_ref[...] + (q_ref[0] + q_ref[1]) * dinv_ref[...]


_tc3 = pl.pallas_call(
    _tc3_body,
    grid=(GRID,),
    in_specs=[
        pl.BlockSpec((BM, D), lambda i: (i, 0)),
        pl.BlockSpec((NC, BM, D), lambda i: (0, i, 0)),
        pl.BlockSpec((BM, 1), lambda i: (i, 0)),
    ],
    out_specs=pl.BlockSpec((BM, D), lambda i: (i, 0)),
    out_shape=jax.ShapeDtypeStruct((NP, D), jnp.float32),
)


@jax.jit
def kernel(id_embedding, edge_index, s, W1, W2):
    src = edge_index[0]
    dst = edge_index[1]
    xp = jnp.zeros((NP, D), jnp.float32).at[:N].set(id_embedding)
    src3 = src.reshape(NW, EPT)
    dst3 = dst.reshape(NW, EPT)
    s3 = s.reshape(NW, EPT)
    degp = _deg_kernel(src, dst, s)
    h1p, dinv = _tc1(xp, W1, degp)
    p = _agg_kernel(h1p, src3, dst3, s3)
    x1, h2p = _tc2(p, dinv, W2)
    q = _agg_kernel(h2p, src3, dst3, s3)
    out = _tc3(x1, q, dinv)
    return out[:N]


# speedup vs baseline: 20.9576x; 1.3496x over previous
import functools

import jax
import jax.numpy as jnp
from jax import lax
from jax.experimental import pallas as pl
from jax.experimental.pallas import tpu as pltpu
from jax.experimental.pallas import tpu_sc as plsc

N = 10000
E = 320000
D = 128
NP = 10240
NC = 2
NS = 16
NW = NC * NS
EPT = E // NW
RPT = NP // NS

CH = 16
NCHUNK = EPT // CH
NBUF = 5
AHEAD = 3

DCH = 80
DNCHUNK = EPT // DCH

_mesh = plsc.VectorSubcoreMesh(core_axis_name="c", subcore_axis_name="s")


DNB = 5


@functools.partial(
    pl.kernel,
    out_type=jax.ShapeDtypeStruct((NC, NP), jnp.float32),
    mesh=_mesh,
    scratch_types=[
        pltpu.VMEM_SHARED((NP,), jnp.float32),
        pltpu.VMEM((EPT,), jnp.float32),
        [pltpu.VMEM((DCH,), jnp.int32) for _ in range(DNB)],
        [pltpu.SemaphoreType.DMA for _ in range(DNB)],
        [pltpu.SemaphoreType.DMA for _ in range(DNB)],
        pltpu.VMEM((RPT,), jnp.float32),
    ],
)
def _deg_kernel(src, dst, s, out, acc, sa_v, ibuf, isem, ssem, z_v):
    cid = lax.axis_index("c")
    sid = lax.axis_index("s")
    wid = sid * NC + cid

    def zb(i, _):
        z_v[pl.ds(i * 16, 16)] = jnp.zeros((16,), jnp.float32)
        return 0

    lax.fori_loop(0, RPT // 16, zb, 0)
    row0 = pl.multiple_of(sid * RPT, 8)
    pltpu.sync_copy(z_v, acc.at[pl.ds(row0, RPT)])
    ebase = wid * EPT
    pltpu.sync_copy(s.at[pl.ds(pl.multiple_of(ebase, 8), EPT)], sa_v)
    plsc.subcore_barrier()

    def one_pass(idx1):
        def start_idx(j, b):
            off = pl.multiple_of(ebase + j * DCH, 8)
            pltpu.async_copy(idx1.at[pl.ds(off, DCH)], ibuf[b], isem[b])

        for j0 in range(2):
            start_idx(j0, j0)

        def outer(t, _):
            for p in range(DNB):
                i = t * DNB + p
                j = i + 2
                jb = (p + 2) % DNB

                @pl.when(j < DNCHUNK)
                def _():
                    @pl.when(j >= DNB)
                    def _():
                        soff = pl.multiple_of((j - DNB) * DCH, 8)
                        pltpu.make_async_copy(
                            sa_v.at[pl.ds(soff, DCH)], acc.at[ibuf[jb]], ssem[jb]
                        ).wait()

                    start_idx(j, jb)

                pltpu.make_async_copy(
                    idx1.at[pl.ds(pl.multiple_of(ebase + i * DCH, 8), DCH)],
                    ibuf[p],
                    isem[p],
                ).wait()
                voff = pl.multiple_of(i * DCH, 8)
                pltpu.async_copy(
                    sa_v.at[pl.ds(voff, DCH)], acc.at[ibuf[p]], ssem[p], add=True
                )
            return 0

        lax.fori_loop(0, DNCHUNK // DNB, outer, 0)

        for p in range(DNB):
            i_last = DNCHUNK - DNB + p
            soff = pl.multiple_of(i_last * DCH, 8)
            pltpu.make_async_copy(
                sa_v.at[pl.ds(soff, DCH)], acc.at[ibuf[p]], ssem[p]
            ).wait()

    one_pass(dst)
    one_pass(src)

    plsc.subcore_barrier()
    pltpu.sync_copy(acc.at[pl.ds(row0, RPT)], out.at[cid, pl.ds(row0, RPT)])


@functools.partial(
    pl.kernel,
    out_type=jax.ShapeDtypeStruct((NC, NP, D), jnp.float32),
    mesh=_mesh,
    scratch_types=[
        pltpu.VMEM_SHARED((NP, D), jnp.float32),
        pltpu.VMEM((EPT,), jnp.int32),
        pltpu.VMEM((EPT,), jnp.int32),
        pltpu.VMEM((EPT,), jnp.float32),
        [pltpu.VMEM((CH, D), jnp.float32) for _ in range(NBUF)],
        [pltpu.SemaphoreType.DMA for _ in range(NBUF)],
        [pltpu.SemaphoreType.DMA for _ in range(NBUF)],
    ],
)
def _agg_kernel(h, src3, dst3, s3, out, acc, si_v, di_v, sv_v, rows, gsem, ssem):
    cid = lax.axis_index("c")
    sid = lax.axis_index("s")
    wid = sid * NC + cid

    def zb(j, _):
        for c in range(D // 16):
            rows[0][j, pl.ds(c * 16, 16)] = jnp.zeros((16,), jnp.float32)
        return 0

    lax.fori_loop(0, CH, zb, 0)
    nz = RPT // CH
    for p in range(nz):
        pltpu.async_copy(
            rows[0], acc.at[pl.ds(sid * RPT + p * CH, CH)], gsem[p % NBUF]
        )
    for p in range(nz):
        pltpu.make_async_copy(
            rows[0], acc.at[pl.ds(sid * RPT + p * CH, CH)], gsem[p % NBUF]
        ).wait()

    pltpu.sync_copy(src3.at[wid], si_v)
    pltpu.sync_copy(dst3.at[wid], di_v)
    pltpu.sync_copy(s3.at[wid], sv_v)
    plsc.subcore_barrier()

    def start_gather(j, b):
        si = si_v[pl.ds(j * CH, CH)]
        pltpu.async_copy(h.at[si], rows[b], gsem[b])

    for j0 in range(AHEAD):
        start_gather(j0, j0)

    def outer(t, _):
        for p in range(NBUF):
            i = t * NBUF + p
            j = i + AHEAD
            jb = (p + AHEAD) % NBUF

            @pl.when(j < NCHUNK)
            def _():
                @pl.when(j >= NBUF)
                def _():
                    di = di_v[pl.ds(j * CH, CH)]
                    pltpu.make_async_copy(
                        rows[jb], acc.at[di], ssem[jb]
                    ).wait()

                start_gather(j, jb)

            si = si_v[pl.ds(i * CH, CH)]
            pltpu.make_async_copy(h.at[si], rows[p], gsem[p]).wait()

            sv = sv_v[pl.ds(i * CH, CH)]
            for j2 in range(CH):
                b = jnp.broadcast_to(sv[j2], (16,))
                for c in range(D // 16):
                    rows[p][j2, pl.ds(c * 16, 16)] = (
                        rows[p][j2, pl.ds(c * 16, 16)] * b
                    )

            di = di_v[pl.ds(i * CH, CH)]
            pltpu.async_copy(rows[p], acc.at[di], ssem[p], add=True)
        return 0

    lax.fori_loop(0, NCHUNK // NBUF, outer, 0)

    for p in range(NBUF):
        i_last = NCHUNK - NBUF + p
        di = di_v[pl.ds(i_last * CH, CH)]
        pltpu.make_async_copy(rows[p], acc.at[di], ssem[p]).wait()

    plsc.subcore_barrier()
    row0 = pl.multiple_of(sid * RPT, 8)
    pltpu.sync_copy(acc.at[pl.ds(row0, RPT)], out.at[cid, pl.ds(row0, RPT)])


BM = 1024
GRID = NP // BM


def _tc1_body(x_ref, w_ref, degp_ref, h_ref, dinv_ref):
    deg = degp_ref[0, :] + degp_ref[1, :]
    dinv = lax.rsqrt(jnp.maximum(deg, 1e-12))
    h = jnp.dot(x_ref[...], w_ref[...], preferred_element_type=jnp.float32)
    h_ref[...] = h * dinv[:, None]
    dinv_ref[...] = dinv[:, None]


_tc1 = pl.pallas_call(
    _tc1_body,
    grid=(GRID,),
    in_specs=[
        pl.BlockSpec((BM, D), lambda i: (i, 0)),
        pl.BlockSpec((D, D), lambda i: (0, 0)),
        pl.BlockSpec((NC, BM), lambda i: (0, i)),
    ],
    out_specs=[
        pl.BlockSpec((BM, D), lambda i: (i, 0)),
        pl.BlockSpec((BM, 1), lambda i: (i, 0)),
    ],
    out_shape=[
        jax.ShapeDtypeStruct((NP, D), jnp.float32),
        jax.ShapeDtypeStruct((NP, 1), jnp.float32),
    ],
)


def _tc2_body(p_ref, dinv_ref, w_ref, x1_ref, h2_ref):
    x1 = (p_ref[0] + p_ref[1]) * dinv_ref[...]
    x1_ref[...] = x1
    h2_ref[...] = (
        jnp.dot(x1, w_ref[...], preferred_element_type=jnp.float32) * dinv_ref[...]
    )


_tc2 = pl.pallas_call(
    _tc2_body,
    grid=(GRID,),
    in_specs=[
        pl.BlockSpec((NC, BM, D), lambda i: (0, i, 0)),
        pl.BlockSpec((BM, 1), lambda i: (i, 0)),
        pl.BlockSpec((D, D), lambda i: (0, 0)),
    ],
    out_specs=[
        pl.BlockSpec((BM, D), lambda i: (i, 0)),
        pl.BlockSpec((BM, D), lambda i: (i, 0)),
    ],
    out_shape=[
        jax.ShapeDtypeStruct((NP, D), jnp.float32),
        jax.ShapeDtypeStruct((NP, D), jnp.float32),
    ],
)


def _tc3_body(x1_ref, q_ref, dinv_ref, o_ref):
    o_ref[...] = x1_ref[...] + (q_ref[0] + q_ref[1]) * dinv_ref[...]


_tc3 = pl.pallas_call(
    _tc3_body,
    grid=(GRID,),
    in_specs=[
        pl.BlockSpec((BM, D), lambda i: (i, 0)),
        pl.BlockSpec((NC, BM, D), lambda i: (0, i, 0)),
        pl.BlockSpec((BM, 1), lambda i: (i, 0)),
    ],
    out_specs=pl.BlockSpec((BM, D), lambda i: (i, 0)),
    out_shape=jax.ShapeDtypeStruct((NP, D), jnp.float32),
)


@jax.jit
def kernel(id_embedding, edge_index, s, W1, W2):
    src = edge_index[0]
    dst = edge_index[1]
    xp = jnp.zeros((NP, D), jnp.float32).at[:N].set(id_embedding)
    src3 = src.reshape(NW, EPT)
    dst3 = dst.reshape(NW, EPT)
    s3 = s.reshape(NW, EPT)
    degp = _deg_kernel(src, dst, s)
    h1p, dinv = _tc1(xp, W1, degp)
    p = _agg_kernel(h1p, src3, dst3, s3)
    x1, h2p = _tc2(p, dinv, W2)
    q = _agg_kernel(h2p, src3, dst3, s3)
    out = _tc3(x1, q, dinv)
    return out[:N]


# speedup vs baseline: 21.1650x; 1.0099x over previous
import functools

import jax
import jax.numpy as jnp
from jax import lax
from jax.experimental import pallas as pl
from jax.experimental.pallas import tpu as pltpu
from jax.experimental.pallas import tpu_sc as plsc

N = 10000
E = 320000
D = 128
NP = 10240
NC = 2
NS = 16
NW = NC * NS
EPT = E // NW
RPT = NP // NS

CH = 40
NCHUNK = EPT // CH
NBUF = 5
AHEAD = 3

DCH = 80
DNCHUNK = EPT // DCH

_mesh = plsc.VectorSubcoreMesh(core_axis_name="c", subcore_axis_name="s")


DNB = 5


@functools.partial(
    pl.kernel,
    out_type=jax.ShapeDtypeStruct((NC, NP), jnp.float32),
    mesh=_mesh,
    scratch_types=[
        pltpu.VMEM_SHARED((NP,), jnp.float32),
        pltpu.VMEM((EPT,), jnp.float32),
        [pltpu.VMEM((DCH,), jnp.int32) for _ in range(DNB)],
        [pltpu.SemaphoreType.DMA for _ in range(DNB)],
        [pltpu.SemaphoreType.DMA for _ in range(DNB)],
        pltpu.VMEM((RPT,), jnp.float32),
    ],
)
def _deg_kernel(src, dst, s, out, acc, sa_v, ibuf, isem, ssem, z_v):
    cid = lax.axis_index("c")
    sid = lax.axis_index("s")
    wid = sid * NC + cid

    def zb(i, _):
        z_v[pl.ds(i * 16, 16)] = jnp.zeros((16,), jnp.float32)
        return 0

    lax.fori_loop(0, RPT // 16, zb, 0)
    row0 = pl.multiple_of(sid * RPT, 8)
    pltpu.sync_copy(z_v, acc.at[pl.ds(row0, RPT)])
    ebase = wid * EPT
    pltpu.sync_copy(s.at[pl.ds(pl.multiple_of(ebase, 8), EPT)], sa_v)
    plsc.subcore_barrier()

    def one_pass(idx1):
        def start_idx(j, b):
            off = pl.multiple_of(ebase + j * DCH, 8)
            pltpu.async_copy(idx1.at[pl.ds(off, DCH)], ibuf[b], isem[b])

        for j0 in range(2):
            start_idx(j0, j0)

        def outer(t, _):
            for p in range(DNB):
                i = t * DNB + p
                j = i + 2
                jb = (p + 2) % DNB

                @pl.when(j < DNCHUNK)
                def _():
                    @pl.when(j >= DNB)
                    def _():
                        soff = pl.multiple_of((j - DNB) * DCH, 8)
                        pltpu.make_async_copy(
                            sa_v.at[pl.ds(soff, DCH)], acc.at[ibuf[jb]], ssem[jb]
                        ).wait()

                    start_idx(j, jb)

                pltpu.make_async_copy(
                    idx1.at[pl.ds(pl.multiple_of(ebase + i * DCH, 8), DCH)],
                    ibuf[p],
                    isem[p],
                ).wait()
                voff = pl.multiple_of(i * DCH, 8)
                pltpu.async_copy(
                    sa_v.at[pl.ds(voff, DCH)], acc.at[ibuf[p]], ssem[p], add=True
                )
            return 0

        lax.fori_loop(0, DNCHUNK // DNB, outer, 0)

        for p in range(DNB):
            i_last = DNCHUNK - DNB + p
            soff = pl.multiple_of(i_last * DCH, 8)
            pltpu.make_async_copy(
                sa_v.at[pl.ds(soff, DCH)], acc.at[ibuf[p]], ssem[p]
            ).wait()

    one_pass(dst)
    one_pass(src)

    plsc.subcore_barrier()
    pltpu.sync_copy(acc.at[pl.ds(row0, RPT)], out.at[cid, pl.ds(row0, RPT)])


@functools.partial(
    pl.kernel,
    out_type=jax.ShapeDtypeStruct((NC, NP, D), jnp.float32),
    mesh=_mesh,
    scratch_types=[
        pltpu.VMEM_SHARED((NP, D), jnp.float32),
        pltpu.VMEM((EPT,), jnp.int32),
        [pltpu.VMEM((CH,), jnp.int32) for _ in range(NBUF)],
        [pltpu.VMEM((CH,), jnp.float32) for _ in range(NBUF)],
        [pltpu.VMEM((CH, D), jnp.float32) for _ in range(NBUF)],
        [pltpu.SemaphoreType.DMA for _ in range(NBUF)],
        [pltpu.SemaphoreType.DMA for _ in range(NBUF)],
        [pltpu.SemaphoreType.DMA for _ in range(NBUF)],
        [pltpu.SemaphoreType.DMA for _ in range(NBUF)],
    ],
)
def _agg_kernel(
    h, src3, dst1, s1, out, acc, si_v, dib, svb, rows, dsem, vsem, gsem, ssem
):
    cid = lax.axis_index("c")
    sid = lax.axis_index("s")
    wid = sid * NC + cid

    def zb(j, _):
        for c in range(D // 16):
            rows[0][j, pl.ds(c * 16, 16)] = jnp.zeros((16,), jnp.float32)
        return 0

    lax.fori_loop(0, CH, zb, 0)
    nz = RPT // CH
    for p in range(nz):
        pltpu.async_copy(
            rows[0], acc.at[pl.ds(sid * RPT + p * CH, CH)], gsem[p % NBUF]
        )
    for p in range(nz):
        pltpu.make_async_copy(
            rows[0], acc.at[pl.ds(sid * RPT + p * CH, CH)], gsem[p % NBUF]
        ).wait()

    pltpu.sync_copy(src3.at[wid], si_v)
    ebase = wid * EPT
    plsc.subcore_barrier()

    def start_gather(j, b):
        soff = pl.multiple_of(ebase + j * CH, 8)
        pltpu.async_copy(s1.at[pl.ds(soff, CH)], svb[b], vsem[b])
        pltpu.async_copy(dst1.at[pl.ds(soff, CH)], dib[b], dsem[b])
        loff = pl.multiple_of(j * CH, 8)
        pltpu.async_copy(h.at[si_v.at[pl.ds(loff, CH)]], rows[b], gsem[b])

    for j0 in range(AHEAD):
        start_gather(j0, j0)

    def outer(t, _):
        for p in range(NBUF):
            i = t * NBUF + p
            j = i + AHEAD
            jb = (p + AHEAD) % NBUF

            @pl.when(j < NCHUNK)
            def _():
                @pl.when(j >= NBUF)
                def _():
                    pltpu.make_async_copy(
                        rows[jb], acc.at[dib[jb]], ssem[jb]
                    ).wait()

                start_gather(j, jb)

            soff = pl.multiple_of(ebase + i * CH, 8)
            pltpu.make_async_copy(s1.at[pl.ds(soff, CH)], svb[p], vsem[p]).wait()
            pltpu.make_async_copy(dst1.at[pl.ds(soff, CH)], dib[p], dsem[p]).wait()
            loff = pl.multiple_of(i * CH, 8)
            pltpu.make_async_copy(
                h.at[si_v.at[pl.ds(loff, CH)]], rows[p], gsem[p]
            ).wait()

            for goff, lane_lo, row_base in ((0, 0, 0), (16, 0, 16), (24, 8, 24)):
                sv = svb[p][pl.ds(goff, 16)]
                for j2 in range(lane_lo, 16):
                    r = row_base + j2
                    b = jnp.broadcast_to(sv[j2], (16,))
                    for c in range(D // 16):
                        rows[p][r, pl.ds(c * 16, 16)] = (
                            rows[p][r, pl.ds(c * 16, 16)] * b
                        )

            pltpu.async_copy(rows[p], acc.at[dib[p]], ssem[p], add=True)
        return 0

    lax.fori_loop(0, NCHUNK // NBUF, outer, 0)

    for p in range(NBUF):
        pltpu.make_async_copy(rows[p], acc.at[dib[p]], ssem[p]).wait()

    plsc.subcore_barrier()
    row0 = pl.multiple_of(sid * RPT, 8)
    pltpu.sync_copy(acc.at[pl.ds(row0, RPT)], out.at[cid, pl.ds(row0, RPT)])


BM = 1024
GRID = NP // BM


def _tc1_body(x_ref, w_ref, degp_ref, h_ref, dinv_ref):
    deg = degp_ref[0, :] + degp_ref[1, :]
    dinv = lax.rsqrt(jnp.maximum(deg, 1e-12))
    h = jnp.dot(x_ref[...], w_ref[...], preferred_element_type=jnp.float32)
    h_ref[...] = h * dinv[:, None]
    dinv_ref[...] = dinv[:, None]


_tc1 = pl.pallas_call(
    _tc1_body,
    grid=(GRID,),
    in_specs=[
        pl.BlockSpec((BM, D), lambda i: (i, 0)),
        pl.BlockSpec((D, D), lambda i: (0, 0)),
        pl.BlockSpec((NC, BM), lambda i: (0, i)),
    ],
    out_specs=[
        pl.BlockSpec((BM, D), lambda i: (i, 0)),
        pl.BlockSpec((BM, 1), lambda i: (i, 0)),
    ],
    out_shape=[
        jax.ShapeDtypeStruct((NP, D), jnp.float32),
        jax.ShapeDtypeStruct((NP, 1), jnp.float32),
    ],
)


def _tc2_body(p_ref, dinv_ref, w_ref, x1_ref, h2_ref):
    x1 = (p_ref[0] + p_ref[1]) * dinv_ref[...]
    x1_ref[...] = x1
    h2_ref[...] = (
        jnp.dot(x1, w_ref[...], preferred_element_type=jnp.float32) * dinv_ref[...]
    )


_tc2 = pl.pallas_call(
    _tc2_body,
    grid=(GRID,),
    in_specs=[
        pl.BlockSpec((NC, BM, D), lambda i: (0, i, 0)),
        pl.BlockSpec((BM, 1), lambda i: (i, 0)),
        pl.BlockSpec((D, D), lambda i: (0, 0)),
    ],
    out_specs=[
        pl.BlockSpec((BM, D), lambda i: (i, 0)),
        pl.BlockSpec((BM, D), lambda i: (i, 0)),
    ],
    out_shape=[
        jax.ShapeDtypeStruct((NP, D), jnp.float32),
        jax.ShapeDtypeStruct((NP, D), jnp.float32),
    ],
)


def _tc3_body(x1_ref, q_ref, dinv_ref, o_ref):
    o_ref[...] = x1_ref[...] + (q_ref[0] + q_ref[1]) * dinv_ref[...]


_tc3 = pl.pallas_call(
    _tc3_body,
    grid=(GRID,),
    in_specs=[
        pl.BlockSpec((BM, D), lambda i: (i, 0)),
        pl.BlockSpec((NC, BM, D), lambda i: (0, i, 0)),
        pl.BlockSpec((BM, 1), lambda i: (i, 0)),
    ],
    out_specs=pl.BlockSpec((BM, D), lambda i: (i, 0)),
    out_shape=jax.ShapeDtypeStruct((NP, D), jnp.float32),
)


@jax.jit
def kernel(id_embedding, edge_index, s, W1, W2):
    src = edge_index[0]
    dst = edge_index[1]
    xp = jnp.zeros((NP, D), jnp.float32).at[:N].set(id_embedding)
    src3 = src.reshape(NW, EPT)
    degp = _deg_kernel(src, dst, s)
    h1p, dinv = _tc1(xp, W1, degp)
    p = _agg_kernel(h1p, src3, dst, s)
    x1, h2p = _tc2(p, dinv, W2)
    q = _agg_kernel(h2p, src3, dst, s)
    out = _tc3(x1, q, dinv)
    return out[:N]


# speedup vs baseline: 21.2987x; 1.0063x over previous
import functools

import jax
import jax.numpy as jnp
from jax import lax
from jax.experimental import pallas as pl
from jax.experimental.pallas import tpu as pltpu
from jax.experimental.pallas import tpu_sc as plsc

N = 10000
E = 320000
D = 128
NP = 10240
NC = 2
NS = 16
NW = NC * NS
EPT = E // NW
RPT = NP // NS

CH = 40
NCHUNK = EPT // CH
NBUF = 5
AHEAD = 3

DCH = 80
DNCHUNK = EPT // DCH

_mesh = plsc.VectorSubcoreMesh(core_axis_name="c", subcore_axis_name="s")


DNB = 5


@functools.partial(
    pl.kernel,
    out_type=jax.ShapeDtypeStruct((NC, NP), jnp.float32),
    mesh=_mesh,
    scratch_types=[
        pltpu.VMEM_SHARED((NP,), jnp.float32),
        pltpu.VMEM((EPT,), jnp.float32),
        [pltpu.VMEM((DCH,), jnp.int32) for _ in range(DNB)],
        [pltpu.SemaphoreType.DMA for _ in range(DNB)],
        [pltpu.SemaphoreType.DMA for _ in range(DNB)],
        pltpu.VMEM((RPT,), jnp.float32),
    ],
)
def _deg_kernel(src, dst, s, out, acc, sa_v, ibuf, isem, ssem, z_v):
    cid = lax.axis_index("c")
    sid = lax.axis_index("s")
    wid = sid * NC + cid

    def zb(i, _):
        z_v[pl.ds(i * 16, 16)] = jnp.zeros((16,), jnp.float32)
        return 0

    lax.fori_loop(0, RPT // 16, zb, 0)
    row0 = pl.multiple_of(sid * RPT, 8)
    pltpu.sync_copy(z_v, acc.at[pl.ds(row0, RPT)])
    ebase = wid * EPT
    pltpu.sync_copy(s.at[pl.ds(pl.multiple_of(ebase, 8), EPT)], sa_v)
    plsc.subcore_barrier()

    def one_pass(idx1):
        def start_idx(j, b):
            off = pl.multiple_of(ebase + j * DCH, 8)
            pltpu.async_copy(idx1.at[pl.ds(off, DCH)], ibuf[b], isem[b])

        for j0 in range(2):
            start_idx(j0, j0)

        def outer(t, _):
            for p in range(DNB):
                i = t * DNB + p
                j = i + 2
                jb = (p + 2) % DNB

                @pl.when(j < DNCHUNK)
                def _():
                    @pl.when(j >= DNB)
                    def _():
                        soff = pl.multiple_of((j - DNB) * DCH, 8)
                        pltpu.make_async_copy(
                            sa_v.at[pl.ds(soff, DCH)], acc.at[ibuf[jb]], ssem[jb]
                        ).wait()

                    start_idx(j, jb)

                pltpu.make_async_copy(
                    idx1.at[pl.ds(pl.multiple_of(ebase + i * DCH, 8), DCH)],
                    ibuf[p],
                    isem[p],
                ).wait()
                voff = pl.multiple_of(i * DCH, 8)
                pltpu.async_copy(
                    sa_v.at[pl.ds(voff, DCH)], acc.at[ibuf[p]], ssem[p], add=True
                )
            return 0

        lax.fori_loop(0, DNCHUNK // DNB, outer, 0)

        for p in range(DNB):
            i_last = DNCHUNK - DNB + p
            soff = pl.multiple_of(i_last * DCH, 8)
            pltpu.make_async_copy(
                sa_v.at[pl.ds(soff, DCH)], acc.at[ibuf[p]], ssem[p]
            ).wait()

    one_pass(dst)
    one_pass(src)

    plsc.subcore_barrier()
    pltpu.sync_copy(acc.at[pl.ds(row0, RPT)], out.at[cid, pl.ds(row0, RPT)])


@functools.partial(
    pl.kernel,
    out_type=jax.ShapeDtypeStruct((NC, NP, D), jnp.float32),
    mesh=_mesh,
    scratch_types=[
        pltpu.VMEM_SHARED((NP, D), jnp.float32),
        pltpu.VMEM((EPT,), jnp.int32),
        [pltpu.VMEM((CH,), jnp.int32) for _ in range(NBUF)],
        [pltpu.VMEM((CH,), jnp.float32) for _ in range(NBUF)],
        [pltpu.VMEM((CH, D), jnp.float32) for _ in range(NBUF)],
        [pltpu.SemaphoreType.DMA for _ in range(NBUF)],
        [pltpu.SemaphoreType.DMA for _ in range(NBUF)],
        [pltpu.SemaphoreType.DMA for _ in range(NBUF)],
        [pltpu.SemaphoreType.DMA for _ in range(NBUF)],
    ],
)
def _agg_kernel(
    h, src3, dst1, s1, out, acc, si_v, dib, svb, rows, dsem, vsem, gsem, ssem
):
    cid = lax.axis_index("c")
    sid = lax.axis_index("s")
    wid = sid * NC + cid

    def zb(j, _):
        for c in range(D // 16):
            rows[0][j, pl.ds(c * 16, 16)] = jnp.zeros((16,), jnp.float32)
        return 0

    lax.fori_loop(0, CH, zb, 0)
    nz = RPT // CH
    for p in range(nz):
        pltpu.async_copy(
            rows[0], acc.at[pl.ds(sid * RPT + p * CH, CH)], gsem[p % NBUF]
        )
    for p in range(nz):
        pltpu.make_async_copy(
            rows[0], acc.at[pl.ds(sid * RPT + p * CH, CH)], gsem[p % NBUF]
        ).wait()

    pltpu.sync_copy(src3.at[wid], si_v)
    ebase = wid * EPT
    plsc.subcore_barrier()

    def start_gather(j, b):
        soff = pl.multiple_of(ebase + j * CH, 8)
        pltpu.async_copy(s1.at[pl.ds(soff, CH)], svb[b], vsem[b])
        pltpu.async_copy(dst1.at[pl.ds(soff, CH)], dib[b], dsem[b])
        loff = pl.multiple_of(j * CH, 8)
        pltpu.async_copy(h.at[si_v.at[pl.ds(loff, CH)]], rows[b], gsem[b])

    for j0 in range(AHEAD):
        start_gather(j0, j0)

    def outer(t, _):
        for p in range(NBUF):
            i = t * NBUF + p
            j = i + AHEAD
            jb = (p + AHEAD) % NBUF

            @pl.when(j < NCHUNK)
            def _():
                @pl.when(j >= NBUF)
                def _():
                    pltpu.make_async_copy(
                        rows[jb], acc.at[dib[jb]], ssem[jb]
                    ).wait()

                start_gather(j, jb)

            soff = pl.multiple_of(ebase + i * CH, 8)
            pltpu.make_async_copy(s1.at[pl.ds(soff, CH)], svb[p], vsem[p]).wait()
            pltpu.make_async_copy(dst1.at[pl.ds(soff, CH)], dib[p], dsem[p]).wait()
            loff = pl.multiple_of(i * CH, 8)
            pltpu.make_async_copy(
                h.at[si_v.at[pl.ds(loff, CH)]], rows[p], gsem[p]
            ).wait()

            for goff, lane_lo, row_base in ((0, 0, 0), (16, 0, 16), (24, 8, 24)):
                sv = svb[p][pl.ds(goff, 16)]
                for j2 in range(lane_lo, 16):
                    r = row_base + j2
                    b = jnp.broadcast_to(sv[j2], (16,))
                    for c in range(D // 16):
                        rows[p][r, pl.ds(c * 16, 16)] = (
                            rows[p][r, pl.ds(c * 16, 16)] * b
                        )

            pltpu.async_copy(rows[p], acc.at[dib[p]], ssem[p], add=True, priority=1)
        return 0

    lax.fori_loop(0, NCHUNK // NBUF, outer, 0)

    for p in range(NBUF):
        pltpu.make_async_copy(rows[p], acc.at[dib[p]], ssem[p]).wait()

    plsc.subcore_barrier()
    row0 = pl.multiple_of(sid * RPT, 8)
    pltpu.sync_copy(acc.at[pl.ds(row0, RPT)], out.at[cid, pl.ds(row0, RPT)])


BM = 1024
GRID = NP // BM


def _tc1_body(x_ref, w_ref, degp_ref, h_ref, dinv_ref):
    deg = degp_ref[0, :] + degp_ref[1, :]
    dinv = lax.rsqrt(jnp.maximum(deg, 1e-12))
    h = jnp.dot(x_ref[...], w_ref[...], preferred_element_type=jnp.float32)
    h_ref[...] = h * dinv[:, None]
    dinv_ref[...] = dinv[:, None]


_tc1 = pl.pallas_call(
    _tc1_body,
    grid=(GRID,),
    in_specs=[
        pl.BlockSpec((BM, D), lambda i: (i, 0)),
        pl.BlockSpec((D, D), lambda i: (0, 0)),
        pl.BlockSpec((NC, BM), lambda i: (0, i)),
    ],
    out_specs=[
        pl.BlockSpec((BM, D), lambda i: (i, 0)),
        pl.BlockSpec((BM, 1), lambda i: (i, 0)),
    ],
    out_shape=[
        jax.ShapeDtypeStruct((NP, D), jnp.float32),
        jax.ShapeDtypeStruct((NP, 1), jnp.float32),
    ],
)


def _tc2_body(p_ref, dinv_ref, w_ref, x1_ref, h2_ref):
    x1 = (p_ref[0] + p_ref[1]) * dinv_ref[...]
    x1_ref[...] = x1
    h2_ref[...] = (
        jnp.dot(x1, w_ref[...], preferred_element_type=jnp.float32) * dinv_ref[...]
    )


_tc2 = pl.pallas_call(
    _tc2_body,
    grid=(GRID,),
    in_specs=[
        pl.BlockSpec((NC, BM, D), lambda i: (0, i, 0)),
        pl.BlockSpec((BM, 1), lambda i: (i, 0)),
        pl.BlockSpec((D, D), lambda i: (0, 0)),
    ],
    out_specs=[
        pl.BlockSpec((BM, D), lambda i: (i, 0)),
        pl.BlockSpec((BM, D), lambda i: (i, 0)),
    ],
    out_shape=[
        jax.ShapeDtypeStruct((NP, D), jnp.float32),
        jax.ShapeDtypeStruct((NP, D), jnp.float32),
    ],
)


def _tc3_body(x1_ref, q_ref, dinv_ref, o_ref):
    o_ref[...] = x1_ref[...] + (q_ref[0] + q_ref[1]) * dinv_ref[...]


_tc3 = pl.pallas_call(
    _tc3_body,
    grid=(GRID,),
    in_specs=[
        pl.BlockSpec((BM, D), lambda i: (i, 0)),
        pl.BlockSpec((NC, BM, D), lambda i: (0, i, 0)),
        pl.BlockSpec((BM, 1), lambda i: (i, 0)),
    ],
    out_specs=pl.BlockSpec((BM, D), lambda i: (i, 0)),
    out_shape=jax.ShapeDtypeStruct((NP, D), jnp.float32),
)


@jax.jit
def kernel(id_embedding, edge_index, s, W1, W2):
    src = edge_index[0]
    dst = edge_index[1]
    xp = jnp.zeros((NP, D), jnp.float32).at[:N].set(id_embedding)
    src3 = src.reshape(NW, EPT)
    degp = _deg_kernel(src, dst, s)
    h1p, dinv = _tc1(xp, W1, degp)
    p = _agg_kernel(h1p, src3, dst, s)
    x1, h2p = _tc2(p, dinv, W2)
    q = _agg_kernel(h2p, src3, dst, s)
    out = _tc3(x1, q, dinv)
    return out[:N]
